# baseline (speedup 1.0000x reference)
import functools

import jax
import jax.numpy as jnp
from jax import lax
from jax.experimental import pallas as pl
from jax.experimental.pallas import tpu as pltpu

NUM_EXPERTS = 8
TOP_K = 2
DIM = 768
T = 2048


def _moe_body(x_ref, wr_ref, br_ref, we_ref, be_ref, out_ref, aux_ref,
              gate_ref):
    e = pl.program_id(0)

    @pl.when(e == 0)
    def _router():
        x = x_ref[...]
        logits = jnp.dot(x, wr_ref[...],
                         preferred_element_type=jnp.float32) + br_ref[...]
        mx = jnp.max(logits, axis=1, keepdims=True)
        ex = jnp.exp(logits - mx)
        probs = ex / jnp.sum(ex, axis=1, keepdims=True)

        iota = lax.broadcasted_iota(jnp.int32, (T, NUM_EXPERTS), 1)
        m1 = jnp.max(probs, axis=1, keepdims=True)
        a1 = jnp.min(jnp.where(probs == m1, iota, NUM_EXPERTS), axis=1,
                     keepdims=True)
        sel1 = iota == a1
        probs_rest = jnp.where(sel1, -1.0, probs)
        m2 = jnp.max(probs_rest, axis=1, keepdims=True)
        a2 = jnp.min(jnp.where(probs_rest == m2, iota, NUM_EXPERTS), axis=1,
                     keepdims=True)
        sel2 = iota == a2

        gate = jnp.where(sel1, m1, 0.0) + jnp.where(sel2, m2, 0.0)
        gate_ref[...] = gate

        f = jnp.sum(sel1.astype(jnp.float32) + sel2.astype(jnp.float32),
                    axis=0) / (T * TOP_K)
        P = jnp.mean(probs, axis=0)
        aux_ref[0, 0] = NUM_EXPERTS * jnp.sum(f * P)

        out_ref[...] = jnp.dot(gate, be_ref[...],
                               preferred_element_type=jnp.float32)

    iota = lax.broadcasted_iota(jnp.int32, (T, NUM_EXPERTS), 1)
    g_e = jnp.sum(jnp.where(iota == e, gate_ref[...], 0.0), axis=1,
                  keepdims=True)
    out_ref[...] += g_e * jnp.dot(x_ref[...].astype(jnp.bfloat16),
                                  we_ref[0].astype(jnp.bfloat16),
                                  preferred_element_type=jnp.float32)


@jax.jit
def kernel(X, Wr, br, We, be):
    Xf = X.reshape(T, DIM)
    br2 = br.reshape(1, NUM_EXPERTS)

    out, aux = pl.pallas_call(
        _moe_body,
        grid=(NUM_EXPERTS,),
        in_specs=[
            pl.BlockSpec((T, DIM), lambda e: (0, 0)),
            pl.BlockSpec((DIM, NUM_EXPERTS), lambda e: (0, 0)),
            pl.BlockSpec((1, NUM_EXPERTS), lambda e: (0, 0)),
            pl.BlockSpec((1, DIM, DIM), lambda e: (e, 0, 0)),
            pl.BlockSpec((NUM_EXPERTS, DIM), lambda e: (0, 0)),
        ],
        out_specs=[
            pl.BlockSpec((T, DIM), lambda e: (0, 0)),
            pl.BlockSpec(memory_space=pltpu.SMEM),
        ],
        out_shape=[
            jax.ShapeDtypeStruct((T, DIM), jnp.float32),
            jax.ShapeDtypeStruct((1, 1), jnp.float32),
        ],
        scratch_shapes=[pltpu.VMEM((T, NUM_EXPERTS), jnp.float32)],
    )(Xf, Wr, br2, We, be)

    return out.reshape(X.shape), aux[0, 0]


# speedup vs baseline: 1.0106x; 1.0106x over previous
import functools

import jax
import jax.numpy as jnp
from jax import lax
from jax.experimental import pallas as pl
from jax.experimental.pallas import tpu as pltpu

NUM_EXPERTS = 8
TOP_K = 2
DIM = 768
T = 2048


def _moe_body(x_ref, wr_ref, br_ref, we_ref, be_ref, out_ref, aux_ref,
              gate_ref):
    e = pl.program_id(0)

    @pl.when(e == 0)
    def _router():
        x = x_ref[...]
        logits = jnp.dot(x, wr_ref[...],
                         preferred_element_type=jnp.float32) + br_ref[...]
        mx = jnp.max(logits, axis=1, keepdims=True)
        ex = jnp.exp(logits - mx)
        probs = ex / jnp.sum(ex, axis=1, keepdims=True)

        iota = lax.broadcasted_iota(jnp.int32, (T, NUM_EXPERTS), 1)
        m1 = jnp.max(probs, axis=1, keepdims=True)
        a1 = jnp.min(jnp.where(probs == m1, iota, NUM_EXPERTS), axis=1,
                     keepdims=True)
        sel1 = iota == a1
        probs_rest = jnp.where(sel1, -1.0, probs)
        m2 = jnp.max(probs_rest, axis=1, keepdims=True)
        a2 = jnp.min(jnp.where(probs_rest == m2, iota, NUM_EXPERTS), axis=1,
                     keepdims=True)
        sel2 = iota == a2

        gate = jnp.where(sel1, m1, 0.0) + jnp.where(sel2, m2, 0.0)
        gate_ref[...] = gate

        f = jnp.sum(sel1.astype(jnp.float32) + sel2.astype(jnp.float32),
                    axis=0) / (T * TOP_K)
        P = jnp.mean(probs, axis=0)
        aux_ref[0, 0] = NUM_EXPERTS * jnp.sum(f * P)

        out_ref[...] = jnp.dot(gate, be_ref[...],
                               preferred_element_type=jnp.float32)

    iota = lax.broadcasted_iota(jnp.int32, (T, NUM_EXPERTS), 1)
    g_e = jnp.sum(jnp.where(iota == e, gate_ref[...], 0.0), axis=1,
                  keepdims=True)
    out_ref[...] += g_e * jnp.dot(x_ref[...], we_ref[0],
                                  preferred_element_type=jnp.float32)


@jax.jit
def kernel(X, Wr, br, We, be):
    Xf = X.reshape(T, DIM)
    br2 = br.reshape(1, NUM_EXPERTS)

    out, aux = pl.pallas_call(
        _moe_body,
        grid=(NUM_EXPERTS,),
        in_specs=[
            pl.BlockSpec((T, DIM), lambda e: (0, 0)),
            pl.BlockSpec((DIM, NUM_EXPERTS), lambda e: (0, 0)),
            pl.BlockSpec((1, NUM_EXPERTS), lambda e: (0, 0)),
            pl.BlockSpec((1, DIM, DIM), lambda e: (e, 0, 0)),
            pl.BlockSpec((NUM_EXPERTS, DIM), lambda e: (0, 0)),
        ],
        out_specs=[
            pl.BlockSpec((T, DIM), lambda e: (0, 0)),
            pl.BlockSpec(memory_space=pltpu.SMEM),
        ],
        out_shape=[
            jax.ShapeDtypeStruct((T, DIM), jnp.float32),
            jax.ShapeDtypeStruct((1, 1), jnp.float32),
        ],
        scratch_shapes=[pltpu.VMEM((T, NUM_EXPERTS), jnp.float32)],
    )(Xf, Wr, br2, We, be)

    return out.reshape(X.shape), aux[0, 0]
